# Initial kernel scaffold; baseline (speedup 1.0000x reference)
import functools

import jax
import jax.numpy as jnp
from jax import lax
from jax.experimental import pallas as pl
from jax.experimental.pallas import tpu as pltpu
from jax.experimental.pallas import tpu_sc as plsc

N = 10000
E = 320000
D = 128

NC = 2
NS = 16
LANES = 16

CHUNK = 128
CHUNKS_PER_TILE = 79
EDGES_PER_TILE = CHUNK * CHUNKS_PER_TILE
E_PAD = EDGES_PER_TILE * NC * NS
EDGES_PER_CORE = EDGES_PER_TILE * NS

TRASH = N
NP_ACC = 10240
ROWS_PER_TILE = NP_ACC // NS
NP_CNT = 10016
CNT_ROWS_PER_TILE = NP_CNT // NS

_mesh = plsc.VectorSubcoreMesh(core_axis_name="c", subcore_axis_name="s")


def _pre_body(src_hbm, dst_hbm, dstr_hbm, cnt_s_hbm, cnt_d_hbm,
              sbuf, dbuf, sredir, dredir, ones_v, zcnt, cnt_s_acc, cnt_d_acc):
    c = lax.axis_index("c")
    t = lax.axis_index("s")

    def zinit(i, _):
        zcnt[pl.ds(i, 1), :] = jnp.zeros((1, LANES), jnp.float32)
        return 0
    lax.fori_loop(0, CNT_ROWS_PER_TILE, zinit, 0)

    def oinit(i, _):
        ones_v[pl.ds(i, 1), :] = jnp.ones((1, LANES), jnp.float32)
        return 0
    lax.fori_loop(0, CHUNK, oinit, 0)

    csl = pl.ds(t * CNT_ROWS_PER_TILE, CNT_ROWS_PER_TILE)
    pltpu.sync_copy(zcnt, cnt_s_acc.at[csl])
    pltpu.sync_copy(zcnt, cnt_d_acc.at[csl])
    plsc.subcore_barrier()

    tile_base = c * EDGES_PER_CORE + t * EDGES_PER_TILE

    def chunk_body(j, _):
        base = tile_base + j * CHUNK
        pltpu.sync_copy(src_hbm.at[pl.ds(base, CHUNK)], sbuf)
        pltpu.sync_copy(dst_hbm.at[pl.ds(base, CHUNK)], dbuf)
        for i in range(CHUNK // LANES):
            sl = pl.ds(i * LANES, LANES)
            s = sbuf[sl]
            d = dbuf[sl]
            m = s == d
            trash = jnp.full((LANES,), TRASH, jnp.int32)
            sredir[sl] = jnp.where(m, trash, s)
            dredir[sl] = jnp.where(m, trash, d)
        pltpu.sync_copy(dredir, dstr_hbm.at[pl.ds(base, CHUNK)])
        pltpu.sync_copy(ones_v, cnt_s_acc.at[sredir], add=True)
        pltpu.sync_copy(ones_v, cnt_d_acc.at[dredir], add=True)
        return 0

    lax.fori_loop(0, CHUNKS_PER_TILE, chunk_body, 0)
    plsc.subcore_barrier()

    pltpu.sync_copy(cnt_s_acc.at[csl], cnt_s_hbm.at[c, csl])
    pltpu.sync_copy(cnt_d_acc.at[csl], cnt_d_hbm.at[c, csl])


_preprocess = functools.partial(
    pl.kernel,
    out_type=(
        jax.ShapeDtypeStruct((E_PAD,), jnp.int32),
        jax.ShapeDtypeStruct((NC, NP_CNT, LANES), jnp.float32),
        jax.ShapeDtypeStruct((NC, NP_CNT, LANES), jnp.float32),
    ),
    mesh=_mesh,
    scratch_types=[
        pltpu.VMEM((CHUNK,), jnp.int32),
        pltpu.VMEM((CHUNK,), jnp.int32),
        pltpu.VMEM((CHUNK,), jnp.int32),
        pltpu.VMEM((CHUNK,), jnp.int32),
        pltpu.VMEM((CHUNK, LANES), jnp.float32),
        pltpu.VMEM((CNT_ROWS_PER_TILE, LANES), jnp.float32),
        pltpu.VMEM_SHARED((NP_CNT, LANES), jnp.float32),
        pltpu.VMEM_SHARED((NP_CNT, LANES), jnp.float32),
    ],
)(_pre_body)


def _spmm_body(h_hbm, src_hbm, dstr_hbm, zrows_hbm, out_hbm,
               sbuf, dbuf, rows, acc, sem):
    c = lax.axis_index("c")
    t = lax.axis_index("s")

    rsl = pl.ds(t * ROWS_PER_TILE, ROWS_PER_TILE)
    pltpu.sync_copy(zrows_hbm, acc.at[rsl])
    plsc.subcore_barrier()

    tile_base = c * EDGES_PER_CORE + t * EDGES_PER_TILE

    def chunk_body(j, _):
        base = tile_base + j * CHUNK
        pltpu.sync_copy(src_hbm.at[pl.ds(base, CHUNK)], sbuf)
        pltpu.sync_copy(dstr_hbm.at[pl.ds(base, CHUNK)], dbuf)
        pltpu.async_copy(h_hbm.at[sbuf], rows, sem).wait()
        pltpu.sync_copy(rows, acc.at[dbuf], add=True)
        return 0

    lax.fori_loop(0, CHUNKS_PER_TILE, chunk_body, 0)
    plsc.subcore_barrier()

    pltpu.sync_copy(acc.at[rsl], out_hbm.at[c, rsl])


_spmm = functools.partial(
    pl.kernel,
    out_type=jax.ShapeDtypeStruct((NC, NP_ACC, D), jnp.float32),
    mesh=_mesh,
    scratch_types=[
        pltpu.VMEM((CHUNK,), jnp.int32),
        pltpu.VMEM((CHUNK,), jnp.int32),
        pltpu.VMEM((CHUNK, D), jnp.float32),
        pltpu.VMEM_SHARED((NP_ACC, D), jnp.float32),
        pltpu.SemaphoreType.DMA,
    ],
)(_spmm_body)


ROW_BLK = 1000


def _prologue_body(x_ref, cnt_s_ref, o_ref):
    cnt = cnt_s_ref[0, :, 0:1] + cnt_s_ref[1, :, 0:1]
    ns = lax.rsqrt(cnt + 1.0)
    o_ref[...] = x_ref[...] * ns


def _tc_prologue(x, cnt_s):
    return pl.pallas_call(
        _prologue_body,
        grid=(N // ROW_BLK,),
        in_specs=[
            pl.BlockSpec((ROW_BLK, D), lambda i: (i, 0)),
            pl.BlockSpec((NC, ROW_BLK, LANES), lambda i: (0, i, 0)),
        ],
        out_specs=pl.BlockSpec((ROW_BLK, D), lambda i: (i, 0)),
        out_shape=jax.ShapeDtypeStruct((N, D), jnp.float32),
    )(x, cnt_s)


def _layer_body(agg_ref, hs_ref, cnt_s_ref, cnt_d_ref, w_ref, b_ref, o_ref,
                *, relu, scale_out):
    cnt_d = cnt_d_ref[0, :, 0:1] + cnt_d_ref[1, :, 0:1]
    nd = lax.rsqrt(cnt_d + 1.0)
    agg = agg_ref[0] + agg_ref[1] + hs_ref[...]
    z = agg * nd
    y = jnp.dot(z, w_ref[...], preferred_element_type=jnp.float32) + b_ref[...]
    if relu:
        y = jnp.maximum(y, 0.0)
    if scale_out:
        cnt_s = cnt_s_ref[0, :, 0:1] + cnt_s_ref[1, :, 0:1]
        y = y * lax.rsqrt(cnt_s + 1.0)
    o_ref[...] = y


def _tc_layer(agg, hs, cnt_s, cnt_d, w, b, relu, scale_out):
    body = functools.partial(_layer_body, relu=relu, scale_out=scale_out)
    return pl.pallas_call(
        body,
        grid=(N // ROW_BLK,),
        in_specs=[
            pl.BlockSpec((NC, ROW_BLK, D), lambda i: (0, i, 0)),
            pl.BlockSpec((ROW_BLK, D), lambda i: (i, 0)),
            pl.BlockSpec((NC, ROW_BLK, LANES), lambda i: (0, i, 0)),
            pl.BlockSpec((NC, ROW_BLK, LANES), lambda i: (0, i, 0)),
            pl.BlockSpec((D, D), lambda i: (0, 0)),
            pl.BlockSpec((1, D), lambda i: (0, 0)),
        ],
        out_specs=pl.BlockSpec((ROW_BLK, D), lambda i: (i, 0)),
        out_shape=jax.ShapeDtypeStruct((N, D), jnp.float32),
    )(agg, hs, cnt_s, cnt_d, w, b)


def kernel(x, edge_index, W0, b0, W1, b1, W2, b2):
    src = edge_index[0]
    dst = edge_index[1]
    pad = jnp.zeros((E_PAD - E,), jnp.int32)
    srcp = jnp.concatenate([src, pad])
    dstp = jnp.concatenate([dst, pad])
    zrows = jnp.zeros((ROWS_PER_TILE, D), jnp.float32)

    dstr, cnt_s, cnt_d = _preprocess(srcp, dstp)

    h0s = _tc_prologue(x, cnt_s)
    agg0 = _spmm(h0s, srcp, dstr, zrows)
    h1s = _tc_layer(agg0, h0s, cnt_s, cnt_d, W0, b0.reshape(1, D), True, True)
    agg1 = _spmm(h1s, srcp, dstr, zrows)
    h2s = _tc_layer(agg1, h1s, cnt_s, cnt_d, W1, b1.reshape(1, D), True, True)
    agg2 = _spmm(h2s, srcp, dstr, zrows)
    out = _tc_layer(agg2, h2s, cnt_s, cnt_d, W2, b2.reshape(1, D), False, False)
    return out


# speedup vs baseline: 3.5856x; 3.5856x over previous
import functools

import jax
import jax.numpy as jnp
from jax import lax
from jax.experimental import pallas as pl
from jax.experimental.pallas import tpu as pltpu
from jax.experimental.pallas import tpu_sc as plsc

N = 10000
E = 320000
D = 128

NC = 2
NS = 16
LANES = 16

CHUNK = 128
CHUNKS_PER_TILE = 79
EDGES_PER_TILE = CHUNK * CHUNKS_PER_TILE
E_PAD = EDGES_PER_TILE * NC * NS
EDGES_PER_CORE = EDGES_PER_TILE * NS

TRASH = N
NP_ACC = 10240
ROWS_PER_TILE = NP_ACC // NS

_mesh = plsc.VectorSubcoreMesh(core_axis_name="c", subcore_axis_name="s")


def _pre_body(src_hbm, dst_hbm, sredir_hbm, dredir_hbm,
              sbuf, dbuf, sredir, dredir):
    c = lax.axis_index("c")
    t = lax.axis_index("s")
    tile_base = c * EDGES_PER_CORE + t * EDGES_PER_TILE

    def chunk_body(j, _):
        base = tile_base + j * CHUNK
        pltpu.sync_copy(src_hbm.at[pl.ds(base, CHUNK)], sbuf)
        pltpu.sync_copy(dst_hbm.at[pl.ds(base, CHUNK)], dbuf)
        for i in range(CHUNK // LANES):
            sl = pl.ds(i * LANES, LANES)
            s = sbuf[sl]
            d = dbuf[sl]
            m = s == d
            trash = jnp.full((LANES,), TRASH, jnp.int32)
            sredir[sl] = jnp.where(m, trash, s)
            dredir[sl] = jnp.where(m, trash, d)
        pltpu.sync_copy(sredir, sredir_hbm.at[pl.ds(base, CHUNK)])
        pltpu.sync_copy(dredir, dredir_hbm.at[pl.ds(base, CHUNK)])
        return 0

    lax.fori_loop(0, CHUNKS_PER_TILE, chunk_body, 0)


_preprocess = functools.partial(
    pl.kernel,
    out_type=(
        jax.ShapeDtypeStruct((E_PAD,), jnp.int32),
        jax.ShapeDtypeStruct((E_PAD,), jnp.int32),
    ),
    mesh=_mesh,
    scratch_types=[
        pltpu.VMEM((CHUNK,), jnp.int32),
        pltpu.VMEM((CHUNK,), jnp.int32),
        pltpu.VMEM((CHUNK,), jnp.int32),
        pltpu.VMEM((CHUNK,), jnp.int32),
    ],
)(_pre_body)


def _spmm_body(h_hbm, src_hbm, dstr_hbm, zrows_hbm, out_hbm,
               sbuf, dbuf, rows, acc, sem):
    c = lax.axis_index("c")
    t = lax.axis_index("s")

    rsl = pl.ds(t * ROWS_PER_TILE, ROWS_PER_TILE)
    pltpu.sync_copy(zrows_hbm, acc.at[rsl])
    plsc.subcore_barrier()

    tile_base = c * EDGES_PER_CORE + t * EDGES_PER_TILE

    def chunk_body(j, _):
        base = tile_base + j * CHUNK
        pltpu.sync_copy(src_hbm.at[pl.ds(base, CHUNK)], sbuf)
        pltpu.sync_copy(dstr_hbm.at[pl.ds(base, CHUNK)], dbuf)
        pltpu.async_copy(h_hbm.at[sbuf], rows, sem).wait()
        pltpu.sync_copy(rows, acc.at[dbuf], add=True)
        return 0

    lax.fori_loop(0, CHUNKS_PER_TILE, chunk_body, 0)
    plsc.subcore_barrier()

    pltpu.sync_copy(acc.at[rsl], out_hbm.at[c, rsl])


_spmm = functools.partial(
    pl.kernel,
    out_type=jax.ShapeDtypeStruct((NC, NP_ACC, D), jnp.float32),
    mesh=_mesh,
    scratch_types=[
        pltpu.VMEM((CHUNK,), jnp.int32),
        pltpu.VMEM((CHUNK,), jnp.int32),
        pltpu.VMEM((CHUNK, D), jnp.float32),
        pltpu.VMEM_SHARED((NP_ACC, D), jnp.float32),
        pltpu.SemaphoreType.DMA,
    ],
)(_spmm_body)


ROW_BLK = 1000


def _finalize_body(cnt_ref, o_ref):
    cnt = cnt_ref[0] + cnt_ref[1]
    o_ref[...] = lax.rsqrt(cnt + 1.0)


def _tc_finalize(cnt):
    return pl.pallas_call(
        _finalize_body,
        grid=(N // ROW_BLK,),
        in_specs=[pl.BlockSpec((NC, ROW_BLK, D), lambda i: (0, i, 0))],
        out_specs=pl.BlockSpec((ROW_BLK, D), lambda i: (i, 0)),
        out_shape=jax.ShapeDtypeStruct((N, D), jnp.float32),
    )(cnt)


def _prologue_body(x_ref, ns_ref, o_ref):
    o_ref[...] = x_ref[...] * ns_ref[...]


def _tc_prologue(x, ns):
    return pl.pallas_call(
        _prologue_body,
        grid=(N // ROW_BLK,),
        in_specs=[
            pl.BlockSpec((ROW_BLK, D), lambda i: (i, 0)),
            pl.BlockSpec((ROW_BLK, D), lambda i: (i, 0)),
        ],
        out_specs=pl.BlockSpec((ROW_BLK, D), lambda i: (i, 0)),
        out_shape=jax.ShapeDtypeStruct((N, D), jnp.float32),
    )(x, ns)


def _layer_body(agg_ref, hs_ref, ns_ref, nd_ref, w_ref, b_ref, o_ref,
                *, relu, scale_out):
    agg = agg_ref[0] + agg_ref[1] + hs_ref[...]
    z = agg * nd_ref[...]
    y = jnp.dot(z, w_ref[...], preferred_element_type=jnp.float32) + b_ref[...]
    if relu:
        y = jnp.maximum(y, 0.0)
    if scale_out:
        y = y * ns_ref[...]
    o_ref[...] = y


def _tc_layer(agg, hs, ns, nd, w, b, relu, scale_out):
    body = functools.partial(_layer_body, relu=relu, scale_out=scale_out)
    return pl.pallas_call(
        body,
        grid=(N // ROW_BLK,),
        in_specs=[
            pl.BlockSpec((NC, ROW_BLK, D), lambda i: (0, i, 0)),
            pl.BlockSpec((ROW_BLK, D), lambda i: (i, 0)),
            pl.BlockSpec((ROW_BLK, D), lambda i: (i, 0)),
            pl.BlockSpec((ROW_BLK, D), lambda i: (i, 0)),
            pl.BlockSpec((D, D), lambda i: (0, 0)),
            pl.BlockSpec((1, D), lambda i: (0, 0)),
        ],
        out_specs=pl.BlockSpec((ROW_BLK, D), lambda i: (i, 0)),
        out_shape=jax.ShapeDtypeStruct((N, D), jnp.float32),
    )(agg, hs, ns, nd, w, b)


def kernel(x, edge_index, W0, b0, W1, b1, W2, b2):
    src = edge_index[0]
    dst = edge_index[1]
    pad = jnp.zeros((E_PAD - E,), jnp.int32)
    srcp = jnp.concatenate([src, pad])
    dstp = jnp.concatenate([dst, pad])
    zrows = jnp.zeros((ROWS_PER_TILE, D), jnp.float32)
    ones_feat = jnp.ones((N, D), jnp.float32)

    sredir, dredir = _preprocess(srcp, dstp)

    cnt_d = _spmm(ones_feat, srcp, dredir, zrows)
    cnt_s = _spmm(ones_feat, dstp, sredir, zrows)
    nd = _tc_finalize(cnt_d)
    ns = _tc_finalize(cnt_s)

    h0s = _tc_prologue(x, ns)
    agg0 = _spmm(h0s, srcp, dredir, zrows)
    h1s = _tc_layer(agg0, h0s, ns, nd, W0, b0.reshape(1, D), True, True)
    agg1 = _spmm(h1s, srcp, dredir, zrows)
    h2s = _tc_layer(agg1, h1s, ns, nd, W1, b1.reshape(1, D), True, True)
    agg2 = _spmm(h2s, srcp, dredir, zrows)
    out = _tc_layer(agg2, h2s, ns, nd, W2, b2.reshape(1, D), False, False)
    return out


# speedup vs baseline: 4.2376x; 1.1818x over previous
import functools

import jax
import jax.numpy as jnp
from jax import lax
from jax.experimental import pallas as pl
from jax.experimental.pallas import tpu as pltpu
from jax.experimental.pallas import tpu_sc as plsc

N = 10000
E = 320000
D = 128

NC = 2
NS = 16
LANES = 16

CHUNK = 128
CHUNKS_PER_TILE = 80
EDGES_PER_TILE = CHUNK * CHUNKS_PER_TILE
E_PAD = EDGES_PER_TILE * NC * NS
E_ALLOC = E_PAD + CHUNK
EDGES_PER_CORE = EDGES_PER_TILE * NS

TRASH = N
NP_ACC = 10240
ROWS_PER_TILE = NP_ACC // NS

_mesh = plsc.VectorSubcoreMesh(core_axis_name="c", subcore_axis_name="s")


def _pre_body(src_hbm, dst_hbm, sredir_hbm, dredir_hbm,
              sbuf, dbuf, sredir, dredir):
    c = lax.axis_index("c")
    t = lax.axis_index("s")
    tile_base = c * EDGES_PER_CORE + t * EDGES_PER_TILE

    def chunk_body(j, _):
        base = tile_base + j * CHUNK
        pltpu.sync_copy(src_hbm.at[pl.ds(base, CHUNK)], sbuf)
        pltpu.sync_copy(dst_hbm.at[pl.ds(base, CHUNK)], dbuf)
        for i in range(CHUNK // LANES):
            sl = pl.ds(i * LANES, LANES)
            s = sbuf[sl]
            d = dbuf[sl]
            m = s == d
            trash = jnp.full((LANES,), TRASH, jnp.int32)
            sredir[sl] = jnp.where(m, trash, s)
            dredir[sl] = jnp.where(m, trash, d)
        pltpu.sync_copy(sredir, sredir_hbm.at[pl.ds(base, CHUNK)])
        pltpu.sync_copy(dredir, dredir_hbm.at[pl.ds(base, CHUNK)])
        return 0

    lax.fori_loop(0, CHUNKS_PER_TILE, chunk_body, 0)


_preprocess = functools.partial(
    pl.kernel,
    out_type=(
        jax.ShapeDtypeStruct((E_ALLOC,), jnp.int32),
        jax.ShapeDtypeStruct((E_ALLOC,), jnp.int32),
    ),
    mesh=_mesh,
    scratch_types=[
        pltpu.VMEM((CHUNK,), jnp.int32),
        pltpu.VMEM((CHUNK,), jnp.int32),
        pltpu.VMEM((CHUNK,), jnp.int32),
        pltpu.VMEM((CHUNK,), jnp.int32),
    ],
)(_pre_body)


def _spmm_body(h_hbm, src_hbm, dstr_hbm, zrows_hbm, out_hbm,
               sbuf0, dbuf0, sbuf1, dbuf1, rows0, rows1, acc, gsem0, gsem1):
    c = lax.axis_index("c")
    t = lax.axis_index("s")

    rsl = pl.ds(t * ROWS_PER_TILE, ROWS_PER_TILE)
    pltpu.sync_copy(zrows_hbm, acc.at[rsl])
    plsc.subcore_barrier()

    tile_base = c * EDGES_PER_CORE + t * EDGES_PER_TILE

    def copy_idx(j, sbuf, dbuf):
        base = tile_base + j * CHUNK
        pltpu.sync_copy(src_hbm.at[pl.ds(base, CHUNK)], sbuf)
        pltpu.sync_copy(dstr_hbm.at[pl.ds(base, CHUNK)], dbuf)

    copy_idx(0, sbuf0, dbuf0)
    pltpu.async_copy(h_hbm.at[sbuf0], rows0, gsem0)

    def pair(i, _):
        g = i * 2
        copy_idx(g + 1, sbuf1, dbuf1)
        pltpu.async_copy(h_hbm.at[sbuf1], rows1, gsem1)
        pltpu.make_async_copy(h_hbm.at[sbuf0], rows0, gsem0).wait()
        pltpu.sync_copy(rows0, acc.at[dbuf0], add=True)
        copy_idx(g + 2, sbuf0, dbuf0)
        pltpu.async_copy(h_hbm.at[sbuf0], rows0, gsem0)
        pltpu.make_async_copy(h_hbm.at[sbuf1], rows1, gsem1).wait()
        pltpu.sync_copy(rows1, acc.at[dbuf1], add=True)
        return 0

    lax.fori_loop(0, CHUNKS_PER_TILE // 2, pair, 0)
    pltpu.make_async_copy(h_hbm.at[sbuf0], rows0, gsem0).wait()
    plsc.subcore_barrier()

    pltpu.sync_copy(acc.at[rsl], out_hbm.at[c, rsl])


_spmm = functools.partial(
    pl.kernel,
    out_type=jax.ShapeDtypeStruct((NC, NP_ACC, D), jnp.float32),
    mesh=_mesh,
    scratch_types=[
        pltpu.VMEM((CHUNK,), jnp.int32),
        pltpu.VMEM((CHUNK,), jnp.int32),
        pltpu.VMEM((CHUNK,), jnp.int32),
        pltpu.VMEM((CHUNK,), jnp.int32),
        pltpu.VMEM((CHUNK, D), jnp.float32),
        pltpu.VMEM((CHUNK, D), jnp.float32),
        pltpu.VMEM_SHARED((NP_ACC, D), jnp.float32),
        pltpu.SemaphoreType.DMA,
        pltpu.SemaphoreType.DMA,
    ],
)(_spmm_body)


def _deg_body(sredir_hbm, dredir_hbm, ones_hbm, zrows_hbm, cnts_hbm, cntd_hbm,
              ibuf, ones_v, acc):
    c = lax.axis_index("c")
    t = lax.axis_index("s")
    rsl = pl.ds(t * ROWS_PER_TILE, ROWS_PER_TILE)
    tile_base = c * EDGES_PER_CORE + t * EDGES_PER_TILE

    pltpu.sync_copy(ones_hbm, ones_v)

    for idx_hbm, out_hbm in ((dredir_hbm, cntd_hbm), (sredir_hbm, cnts_hbm)):
        pltpu.sync_copy(zrows_hbm, acc.at[rsl])
        plsc.subcore_barrier()

        def chunk_body(j, _):
            pltpu.sync_copy(idx_hbm.at[pl.ds(tile_base + j * CHUNK, CHUNK)], ibuf)
            pltpu.sync_copy(ones_v, acc.at[ibuf], add=True)
            return 0

        lax.fori_loop(0, CHUNKS_PER_TILE, chunk_body, 0)
        plsc.subcore_barrier()
        pltpu.sync_copy(acc.at[rsl], out_hbm.at[c, rsl])
        plsc.subcore_barrier()


_degrees = functools.partial(
    pl.kernel,
    out_type=(
        jax.ShapeDtypeStruct((NC, NP_ACC, D), jnp.float32),
        jax.ShapeDtypeStruct((NC, NP_ACC, D), jnp.float32),
    ),
    mesh=_mesh,
    scratch_types=[
        pltpu.VMEM((CHUNK,), jnp.int32),
        pltpu.VMEM((CHUNK, D), jnp.float32),
        pltpu.VMEM_SHARED((NP_ACC, D), jnp.float32),
    ],
)(_deg_body)


ROW_BLK = 1000


def _finalize_body(cnt_ref, o_ref):
    cnt = cnt_ref[0] + cnt_ref[1]
    o_ref[...] = lax.rsqrt(cnt + 1.0)


def _tc_finalize(cnt):
    return pl.pallas_call(
        _finalize_body,
        grid=(N // ROW_BLK,),
        in_specs=[pl.BlockSpec((NC, ROW_BLK, D), lambda i: (0, i, 0))],
        out_specs=pl.BlockSpec((ROW_BLK, D), lambda i: (i, 0)),
        out_shape=jax.ShapeDtypeStruct((N, D), jnp.float32),
    )(cnt)


def _prologue_body(x_ref, ns_ref, o_ref):
    o_ref[...] = x_ref[...] * ns_ref[...]


def _tc_prologue(x, ns):
    return pl.pallas_call(
        _prologue_body,
        grid=(N // ROW_BLK,),
        in_specs=[
            pl.BlockSpec((ROW_BLK, D), lambda i: (i, 0)),
            pl.BlockSpec((ROW_BLK, D), lambda i: (i, 0)),
        ],
        out_specs=pl.BlockSpec((ROW_BLK, D), lambda i: (i, 0)),
        out_shape=jax.ShapeDtypeStruct((N, D), jnp.float32),
    )(x, ns)


def _layer_body(agg_ref, hs_ref, ns_ref, nd_ref, w_ref, b_ref, o_ref,
                *, relu, scale_out):
    agg = agg_ref[0] + agg_ref[1] + hs_ref[...]
    z = agg * nd_ref[...]
    y = jnp.dot(z, w_ref[...], preferred_element_type=jnp.float32) + b_ref[...]
    if relu:
        y = jnp.maximum(y, 0.0)
    if scale_out:
        y = y * ns_ref[...]
    o_ref[...] = y


def _tc_layer(agg, hs, ns, nd, w, b, relu, scale_out):
    body = functools.partial(_layer_body, relu=relu, scale_out=scale_out)
    return pl.pallas_call(
        body,
        grid=(N // ROW_BLK,),
        in_specs=[
            pl.BlockSpec((NC, ROW_BLK, D), lambda i: (0, i, 0)),
            pl.BlockSpec((ROW_BLK, D), lambda i: (i, 0)),
            pl.BlockSpec((ROW_BLK, D), lambda i: (i, 0)),
            pl.BlockSpec((ROW_BLK, D), lambda i: (i, 0)),
            pl.BlockSpec((D, D), lambda i: (0, 0)),
            pl.BlockSpec((1, D), lambda i: (0, 0)),
        ],
        out_specs=pl.BlockSpec((ROW_BLK, D), lambda i: (i, 0)),
        out_shape=jax.ShapeDtypeStruct((N, D), jnp.float32),
    )(agg, hs, ns, nd, w, b)


def kernel(x, edge_index, W0, b0, W1, b1, W2, b2):
    src = edge_index[0]
    dst = edge_index[1]
    pad = jnp.zeros((E_ALLOC - E,), jnp.int32)
    srcp = jnp.concatenate([src, pad])
    dstp = jnp.concatenate([dst, pad])
    zrows = jnp.zeros((ROWS_PER_TILE, D), jnp.float32)
    ones_blk = jnp.ones((CHUNK, D), jnp.float32)

    sredir, dredir = _preprocess(srcp, dstp)

    cnt_s, cnt_d = _degrees(sredir, dredir, ones_blk, zrows)
    nd = _tc_finalize(cnt_d)
    ns = _tc_finalize(cnt_s)

    h0s = _tc_prologue(x, ns)
    agg0 = _spmm(h0s, srcp, dredir, zrows)
    h1s = _tc_layer(agg0, h0s, ns, nd, W0, b0.reshape(1, D), True, True)
    agg1 = _spmm(h1s, srcp, dredir, zrows)
    h2s = _tc_layer(agg1, h1s, ns, nd, W1, b1.reshape(1, D), True, True)
    agg2 = _spmm(h2s, srcp, dredir, zrows)
    out = _tc_layer(agg2, h2s, ns, nd, W2, b2.reshape(1, D), False, False)
    return out
